# Initial kernel scaffold; baseline (speedup 1.0000x reference)
import jax
import jax.numpy as jnp
from jax.experimental import pallas as pl


def kernel(x, edge_index, edge_vals, W_enc, b_enc, W, a):
    raise NotImplementedError("write your pallas kernel here")


# speedup vs baseline: 13.7496x; 13.7496x over previous
import functools

import jax
import jax.numpy as jnp
from jax import lax
from jax.experimental import pallas as pl
from jax.experimental.pallas import tpu as pltpu
from jax.experimental.pallas import tpu_sc as plsc

N = 10000
NP = 10240
E = 320000
DF = 128
H = 128
NH = 4
DH = H // NH

NC = 2
NS = 16
EPT = E // (NC * NS)
CHA = 2000
NCHA = EPT // CHA
CHB = 80
NCHB = EPT // CHB
RPT = NP // NS

_f32 = jnp.float32
_i32 = jnp.int32

_SC_PARAMS = pltpu.CompilerParams(needs_layout_passes=False)
_SC_MESH = plsc.VectorSubcoreMesh(core_axis_name="c", subcore_axis_name="s")


def _wts_body(s12_hbm, src_hbm, dst_hbm, vals_hbm,
              w_out, den_out,
              s12_v, srcb, dstb, valsb, wbuf, ibuf, den_sh):
    c = lax.axis_index("c")
    s = lax.axis_index("s")

    pltpu.sync_copy(s12_hbm, s12_v)

    zlen = NP * NH // NS

    def _z(i, _):
        wbuf[pl.ds(i * 16, 16)] = jnp.zeros((16,), _f32)
        return 0
    lax.fori_loop(0, zlen // 16, _z, 0)
    pltpu.sync_copy(wbuf.at[pl.ds(0, zlen)], den_sh.at[pl.ds(s * zlen, zlen)])
    plsc.subcore_barrier()

    ebase = (c * NS + s) * EPT
    lane = lax.iota(_i32, 16)

    def _chunk(t, _):
        base = ebase + t * CHA
        pltpu.sync_copy(src_hbm.at[pl.ds(base, CHA)], srcb)
        pltpu.sync_copy(dst_hbm.at[pl.ds(base, CHA)], dstb)
        pltpu.sync_copy(vals_hbm.at[pl.ds(base, CHA)], valsb)

        def _grp(g, _):
            sv = srcb[pl.ds(g * 16, 16)]
            dv = dstb[pl.ds(g * 16, 16)]
            vv = valsb[pl.ds(g * 16, 16)]
            eid = lane + g * 16
            for k in range(NH):
                s1 = plsc.load_gather(s12_v, [sv * 8 + (2 * k)])
                s2 = plsc.load_gather(s12_v, [dv * 8 + (2 * k + 1)])
                sc = vv * (s1 + s2)
                sc = jnp.where(sc > 0, sc, 0.2 * sc)
                w = jnp.exp(sc)
                pos = eid * NH + k
                plsc.store_scatter(wbuf, [pos], w)
                plsc.store_scatter(ibuf, [pos], sv * NH + k)
            return 0
        lax.fori_loop(0, CHA // 16, _grp, 0)

        pltpu.sync_copy(wbuf, w_out.at[pl.ds(base * NH, CHA * NH)])
        pltpu.sync_copy(wbuf, den_sh.at[ibuf], add=True)
        return 0

    lax.fori_loop(0, NCHA, _chunk, 0)
    plsc.subcore_barrier()
    pltpu.sync_copy(den_sh.at[pl.ds(s * zlen, zlen)],
                    den_out.at[c, pl.ds(s * zlen, zlen)])


_sc_weights = functools.partial(
    pl.kernel,
    out_type=[
        jax.ShapeDtypeStruct((E * NH,), _f32),
        jax.ShapeDtypeStruct((NC, NP * NH), _f32),
    ],
    mesh=_SC_MESH,
    compiler_params=_SC_PARAMS,
    scratch_types=[
        pltpu.VMEM((NP * 8,), _f32),
        pltpu.VMEM((CHA,), _i32),
        pltpu.VMEM((CHA,), _i32),
        pltpu.VMEM((CHA,), _f32),
        pltpu.VMEM((CHA * NH,), _f32),
        pltpu.VMEM((CHA * NH,), _i32),
        pltpu.VMEM_SHARED((NP * NH,), _f32),
    ],
)(_wts_body)


def _agg_body(hh_hbm, src3_hbm, dst_hbm, w_hbm,
              num_out,
              srcall, dstall, wv, rows, num_sh, gsem):
    c = lax.axis_index("c")
    s = lax.axis_index("s")
    ebase = (c * NS + s) * EPT
    row0 = s * RPT

    pltpu.sync_copy(src3_hbm.at[c * NS + s], srcall)
    pltpu.sync_copy(dst_hbm.at[pl.ds(ebase, EPT)], dstall)

    def _z(i, _):
        for j in range(H // 16):
            rows[i, pl.ds(j * 16, 16)] = jnp.zeros((16,), _f32)
        return 0
    lax.fori_loop(0, CHB, _z, 0)
    for j in range(RPT // CHB):
        pltpu.sync_copy(rows, num_sh.at[pl.ds(row0 + j * CHB, CHB)])
    plsc.subcore_barrier()

    def _chunk(t, _):
        base = ebase + t * CHB
        pltpu.async_copy(hh_hbm.at[dstall.at[pl.ds(t * CHB, CHB)]],
                         rows, gsem).wait()
        pltpu.sync_copy(w_hbm.at[pl.ds(base * NH, CHB * NH)], wv)

        def _e(e, _):
            for k in range(NH):
                wk = plsc.load_gather(wv, [jnp.full((16,), e * NH + k, _i32)])
                for h2 in range(DH // 16):
                    off = k * DH + h2 * 16
                    rows[e, pl.ds(off, 16)] = rows[e, pl.ds(off, 16)] * wk
            return 0
        lax.fori_loop(0, CHB, _e, 0)

        pltpu.sync_copy(rows, num_sh.at[srcall.at[t]], add=True)
        return 0

    lax.fori_loop(0, NCHB, _chunk, 0)
    plsc.subcore_barrier()
    pltpu.sync_copy(num_sh.at[pl.ds(row0, RPT)],
                    num_out.at[c, pl.ds(row0, RPT)])


_sc_aggregate = functools.partial(
    pl.kernel,
    out_type=[jax.ShapeDtypeStruct((NC, NP, H), _f32)],
    mesh=_SC_MESH,
    compiler_params=_SC_PARAMS,
    scratch_types=[
        pltpu.VMEM((NCHB, CHB), _i32),
        pltpu.VMEM((EPT,), _i32),
        pltpu.VMEM((CHB * NH,), _f32),
        pltpu.VMEM((CHB, H), _f32),
        pltpu.VMEM_SHARED((NP, H), _f32),
        pltpu.SemaphoreType.DMA,
    ],
)(_agg_body)


ROWB = 1280


def _tc_enc_body(x_ref, we_ref, be_ref, wc_ref, a0_ref, hh_ref, s_ref):
    h = jnp.dot(x_ref[...], we_ref[...], preferred_element_type=_f32) + be_ref[...]
    hh = jnp.dot(h, wc_ref[...], preferred_element_type=_f32)
    hh_ref[...] = hh
    s_ref[...] = jnp.dot(hh, a0_ref[...], preferred_element_type=_f32)


def _tc_mid_body(num_ref, den_ref, sel_ref, wc_ref, a1_ref,
                 h1_ref, hh_ref, s_ref):
    nsum = num_ref[0] + num_ref[1]
    dsum = den_ref[0] + den_ref[1]
    dexp = jnp.dot(dsum, sel_ref[...], preferred_element_type=_f32)
    o = nsum / dexp
    h1 = jnp.where(o > 0, o, jnp.exp(o) - 1.0)
    h1_ref[...] = h1
    hh = jnp.dot(h1, wc_ref[...], preferred_element_type=_f32)
    hh_ref[...] = hh
    s_ref[...] = jnp.dot(hh, a1_ref[...], preferred_element_type=_f32)


def _tc_fin_body(num_ref, den_ref, sel_ref, h1_ref, out_ref):
    nsum = num_ref[0] + num_ref[1]
    dsum = den_ref[0] + den_ref[1]
    dexp = jnp.dot(dsum, sel_ref[...], preferred_element_type=_f32)
    out_ref[...] = nsum / dexp + h1_ref[...]


def _rows_spec(width):
    return pl.BlockSpec((ROWB, width), lambda i: (i, 0))


def _full_spec(r, cols):
    return pl.BlockSpec((r, cols), lambda i: (0, 0))


def _part_spec(width):
    return pl.BlockSpec((NC, ROWB, width), lambda i: (0, i, 0))


_tc_enc = pl.pallas_call(
    _tc_enc_body,
    grid=(NP // ROWB,),
    in_specs=[_rows_spec(DF), _full_spec(DF, H), _full_spec(1, H),
              _full_spec(H, H), _full_spec(H, 2 * NH)],
    out_specs=[_rows_spec(H), _rows_spec(2 * NH)],
    out_shape=[jax.ShapeDtypeStruct((NP, H), _f32),
               jax.ShapeDtypeStruct((NP, 2 * NH), _f32)],
)

_tc_mid = pl.pallas_call(
    _tc_mid_body,
    grid=(NP // ROWB,),
    in_specs=[_part_spec(H), _part_spec(NH), _full_spec(NH, H),
              _full_spec(H, H), _full_spec(H, 2 * NH)],
    out_specs=[_rows_spec(H), _rows_spec(H), _rows_spec(2 * NH)],
    out_shape=[jax.ShapeDtypeStruct((NP, H), _f32),
               jax.ShapeDtypeStruct((NP, H), _f32),
               jax.ShapeDtypeStruct((NP, 2 * NH), _f32)],
)

_tc_fin = pl.pallas_call(
    _tc_fin_body,
    grid=(NP // ROWB,),
    in_specs=[_part_spec(H), _part_spec(NH), _full_spec(NH, H),
              _rows_spec(H)],
    out_specs=_rows_spec(H),
    out_shape=jax.ShapeDtypeStruct((NP, H), _f32),
)


def _build_attn_mat(a_l):
    A = jnp.zeros((H, 2 * NH), _f32)
    for k in range(NH):
        A = A.at[DH * k:DH * (k + 1), 2 * k].set(a_l[k, 0, :DH])
        A = A.at[DH * k:DH * (k + 1), 2 * k + 1].set(a_l[k, 0, DH:])
    return A


def _build_sel():
    S = jnp.zeros((NH, H), _f32)
    for k in range(NH):
        S = S.at[k, DH * k:DH * (k + 1)].set(1.0)
    return S


def _gat_layer_sc(hh, s12, src, src3, dst, vals):
    w, den = _sc_weights(s12.reshape(NP * 8), src, dst, vals)
    (num,) = _sc_aggregate(hh, src3, dst, w)
    return num, den.reshape(NC, NP, NH)


def kernel(x, edge_index, edge_vals, W_enc, b_enc, W, a):
    src = edge_index[0]
    src3 = src.reshape(NC * NS, NCHB, CHB)
    dst = edge_index[1]
    x2 = jnp.pad(x[0], ((0, NP - N), (0, 0)))
    b2 = b_enc.reshape(1, H)
    Wc0 = jnp.transpose(W[0], (1, 0, 2)).reshape(H, H)
    Wc1 = jnp.transpose(W[1], (1, 0, 2)).reshape(H, H)
    A0 = _build_attn_mat(a[0])
    A1 = _build_attn_mat(a[1])
    Sel = _build_sel()

    hh0, s12_0 = _tc_enc(x2, W_enc, b2, Wc0, A0)
    num0, den0 = _gat_layer_sc(hh0, s12_0, src, src3, dst, edge_vals)
    h1, hh1, s12_1 = _tc_mid(num0, den0, Sel, Wc1, A1)
    num1, den1 = _gat_layer_sc(hh1, s12_1, src, src3, dst, edge_vals)
    out = _tc_fin(num1, den1, Sel, h1)
    return out[:N].reshape(N, 1, H)


# speedup vs baseline: 22.3818x; 1.6278x over previous
import functools

import jax
import jax.numpy as jnp
from jax import lax
from jax.experimental import pallas as pl
from jax.experimental.pallas import tpu as pltpu
from jax.experimental.pallas import tpu_sc as plsc

N = 10000
NP = 10240
E = 320000
DF = 128
H = 128
NH = 4
DH = H // NH

NC = 2
NS = 16
EPT = E // (NC * NS)
CHA = 2000
NCHA = EPT // CHA
CHB = 80
NCHB = EPT // CHB
RPT = NP // NS

_f32 = jnp.float32
_i32 = jnp.int32

_SC_PARAMS = pltpu.CompilerParams(needs_layout_passes=False)
_SC_MESH = plsc.VectorSubcoreMesh(core_axis_name="c", subcore_axis_name="s")


def _wts_body(s12_hbm, src_hbm, dst_hbm, vals_hbm,
              w_out, den_out,
              s12_v, srcb, dstb, valsb, wbuf, ibuf, den_sh):
    c = lax.axis_index("c")
    s = lax.axis_index("s")

    pltpu.sync_copy(s12_hbm, s12_v)

    zlen = NP * NH // NS

    def _z(i, _):
        wbuf[pl.ds(i * 16, 16)] = jnp.zeros((16,), _f32)
        return 0
    lax.fori_loop(0, zlen // 16, _z, 0)
    pltpu.sync_copy(wbuf.at[pl.ds(0, zlen)], den_sh.at[pl.ds(s * zlen, zlen)])
    plsc.subcore_barrier()

    ebase = (c * NS + s) * EPT
    lane = lax.iota(_i32, 16)

    def _chunk(t, _):
        base = ebase + t * CHA
        pltpu.sync_copy(src_hbm.at[pl.ds(base, CHA)], srcb)
        pltpu.sync_copy(dst_hbm.at[pl.ds(base, CHA)], dstb)
        pltpu.sync_copy(vals_hbm.at[pl.ds(base, CHA)], valsb)

        def _grp(g, _):
            sv = srcb[pl.ds(g * 16, 16)]
            dv = dstb[pl.ds(g * 16, 16)]
            vv = valsb[pl.ds(g * 16, 16)]
            eid = lane + g * 16
            for k in range(NH):
                s1 = plsc.load_gather(s12_v, [sv * 8 + (2 * k)])
                s2 = plsc.load_gather(s12_v, [dv * 8 + (2 * k + 1)])
                sc = vv * (s1 + s2)
                sc = jnp.where(sc > 0, sc, 0.2 * sc)
                w = jnp.exp(sc)
                pos = eid * NH + k
                plsc.store_scatter(wbuf, [pos], w)
                plsc.store_scatter(ibuf, [pos], sv * NH + k)
            return 0
        lax.fori_loop(0, CHA // 16, _grp, 0)

        pltpu.sync_copy(wbuf, w_out.at[pl.ds(base * NH, CHA * NH)])
        pltpu.sync_copy(wbuf, den_sh.at[ibuf], add=True)
        return 0

    lax.fori_loop(0, NCHA, _chunk, 0)
    plsc.subcore_barrier()
    pltpu.sync_copy(den_sh.at[pl.ds(s * zlen, zlen)],
                    den_out.at[c, pl.ds(s * zlen, zlen)])


_sc_weights = functools.partial(
    pl.kernel,
    out_type=[
        jax.ShapeDtypeStruct((E * NH,), _f32),
        jax.ShapeDtypeStruct((NC, NP * NH), _f32),
    ],
    mesh=_SC_MESH,
    compiler_params=_SC_PARAMS,
    scratch_types=[
        pltpu.VMEM((NP * 8,), _f32),
        pltpu.VMEM((CHA,), _i32),
        pltpu.VMEM((CHA,), _i32),
        pltpu.VMEM((CHA,), _f32),
        pltpu.VMEM((CHA * NH,), _f32),
        pltpu.VMEM((CHA * NH,), _i32),
        pltpu.VMEM_SHARED((NP * NH,), _f32),
    ],
)(_wts_body)


def _agg_body(hh_hbm, src_hbm, dst_hbm, w_hbm,
              num_out,
              dstall, srcb, wv, rows3, num_sh,
              sem_g, sem_s, sem_r, sem_w):
    c = lax.axis_index("c")
    s = lax.axis_index("s")
    wid = c * NS + s
    ebase = wid * EPT
    row0 = s * RPT

    pltpu.sync_copy(dst_hbm.at[pl.ds(ebase, EPT)], dstall)

    def _z(i, _):
        for j in range(H // 16):
            rows3[0, i, pl.ds(j * 16, 16)] = jnp.zeros((16,), _f32)
        return 0
    lax.fori_loop(0, CHB, _z, 0)
    for j in range(RPT // CHB):
        pltpu.sync_copy(rows3.at[0], num_sh.at[pl.ds(row0 + j * CHB, CHB)])

    def _issue_loads(t, j):
        pltpu.async_copy(src_hbm.at[pl.ds(ebase + t * CHB, CHB)],
                         srcb.at[j], sem_r.at[j])
        pltpu.async_copy(
            w_hbm.at[pl.ds((ebase + t * CHB) * NH, CHB * NH)],
            wv.at[pl.ds(j * CHB * NH, CHB * NH)], sem_w.at[j])

    def _wait_loads(t, j):
        pltpu.make_async_copy(src_hbm.at[pl.ds(ebase + t * CHB, CHB)],
                              srcb.at[j], sem_r.at[j]).wait()
        pltpu.make_async_copy(
            w_hbm.at[pl.ds((ebase + t * CHB) * NH, CHB * NH)],
            wv.at[pl.ds(j * CHB * NH, CHB * NH)], sem_w.at[j]).wait()

    def _issue_gather(t, j):
        pltpu.async_copy(hh_hbm.at[dstall.at[pl.ds(t * CHB, CHB)]],
                         rows3.at[j], sem_g.at[j])

    def _wait_gather(t, j):
        pltpu.make_async_copy(hh_hbm.at[dstall.at[pl.ds(t * CHB, CHB)]],
                              rows3.at[j], sem_g.at[j]).wait()

    def _wait_scatter(j):
        pltpu.make_async_copy(rows3.at[j], num_sh.at[srcb.at[j]],
                              sem_s.at[j]).wait()

    def _scale(j):
        wbase = j * CHB * NH

        def _e(e4, _):
            for i in range(4):
                e = e4 * 4 + i
                for k in range(NH):
                    wk = plsc.load_gather(
                        wv, [jnp.full((16,), wbase + e * NH + k, _i32)])
                    for h2 in range(DH // 16):
                        off = k * DH + h2 * 16
                        rows3[j, e, pl.ds(off, 16)] = (
                            rows3[j, e, pl.ds(off, 16)] * wk)
            return 0
        lax.fori_loop(0, CHB // 4, _e, 0)

    def _body(t, j, first=False, last=False):
        j2 = (j + 2) % 3
        _wait_loads(t, j)
        _wait_gather(t, j)
        _scale(j)
        if last:
            pltpu.sync_copy(rows3.at[j], num_sh.at[srcb.at[j]], add=True)
        else:
            pltpu.async_copy(rows3.at[j], num_sh.at[srcb.at[j]],
                             sem_s.at[j], add=True)
        if not first:
            _wait_scatter(j2)
        if not last:
            _issue_loads(t + 2, j2)
            _issue_gather(t + 2, j2)

    _issue_loads(0, 0)
    _issue_loads(1, 1)
    plsc.subcore_barrier()
    _issue_gather(0, 0)
    _issue_gather(1, 1)
    _body(0, 0, first=True)

    def _chunk3(b, _):
        t = 3 * b
        _body(t + 1, 1)
        _body(t + 2, 2)
        _body(t + 3, 0)
        return 0
    lax.fori_loop(0, (NCHB - 5) // 3, _chunk3, 0)

    _body(NCHB - 4, 1)
    _body(NCHB - 3, 2)
    t = NCHB - 2
    _wait_loads(t, 0)
    _wait_gather(t, 0)
    _scale(0)
    pltpu.sync_copy(rows3.at[0], num_sh.at[srcb.at[0]], add=True)
    _wait_scatter(2)
    t = NCHB - 1
    _wait_loads(t, 1)
    _wait_gather(t, 1)
    _scale(1)
    pltpu.sync_copy(rows3.at[1], num_sh.at[srcb.at[1]], add=True)

    plsc.subcore_barrier()
    pltpu.sync_copy(num_sh.at[pl.ds(row0, RPT)],
                    num_out.at[c, pl.ds(row0, RPT)])


_sc_aggregate = functools.partial(
    pl.kernel,
    out_type=[jax.ShapeDtypeStruct((NC, NP, H), _f32)],
    mesh=_SC_MESH,
    compiler_params=_SC_PARAMS,
    scratch_types=[
        pltpu.VMEM((EPT,), _i32),
        pltpu.VMEM((3, CHB), _i32),
        pltpu.VMEM((3 * CHB * NH,), _f32),
        pltpu.VMEM((3, CHB, H), _f32),
        pltpu.VMEM_SHARED((NP, H), _f32),
        pltpu.SemaphoreType.DMA((3,)),
        pltpu.SemaphoreType.DMA((3,)),
        pltpu.SemaphoreType.DMA((3,)),
        pltpu.SemaphoreType.DMA((3,)),
    ],
)(_agg_body)


ROWB = 1280


def _tc_enc_body(x_ref, we_ref, be_ref, wc_ref, a0_ref, hh_ref, s_ref):
    h = jnp.dot(x_ref[...], we_ref[...], preferred_element_type=_f32) + be_ref[...]
    hh = jnp.dot(h, wc_ref[...], preferred_element_type=_f32)
    hh_ref[...] = hh
    s_ref[...] = jnp.dot(hh, a0_ref[...], preferred_element_type=_f32)


def _tc_mid_body(num_ref, den_ref, sel_ref, wc_ref, a1_ref,
                 h1_ref, hh_ref, s_ref):
    nsum = num_ref[0] + num_ref[1]
    dsum = den_ref[0] + den_ref[1]
    dexp = jnp.dot(dsum, sel_ref[...], preferred_element_type=_f32)
    o = nsum / dexp
    h1 = jnp.where(o > 0, o, jnp.exp(o) - 1.0)
    h1_ref[...] = h1
    hh = jnp.dot(h1, wc_ref[...], preferred_element_type=_f32)
    hh_ref[...] = hh
    s_ref[...] = jnp.dot(hh, a1_ref[...], preferred_element_type=_f32)


def _tc_fin_body(num_ref, den_ref, sel_ref, h1_ref, out_ref):
    nsum = num_ref[0] + num_ref[1]
    dsum = den_ref[0] + den_ref[1]
    dexp = jnp.dot(dsum, sel_ref[...], preferred_element_type=_f32)
    out_ref[...] = nsum / dexp + h1_ref[...]


def _rows_spec(width):
    return pl.BlockSpec((ROWB, width), lambda i: (i, 0))


def _full_spec(r, cols):
    return pl.BlockSpec((r, cols), lambda i: (0, 0))


def _part_spec(width):
    return pl.BlockSpec((NC, ROWB, width), lambda i: (0, i, 0))


_tc_enc = pl.pallas_call(
    _tc_enc_body,
    grid=(NP // ROWB,),
    in_specs=[_rows_spec(DF), _full_spec(DF, H), _full_spec(1, H),
              _full_spec(H, H), _full_spec(H, 2 * NH)],
    out_specs=[_rows_spec(H), _rows_spec(2 * NH)],
    out_shape=[jax.ShapeDtypeStruct((NP, H), _f32),
               jax.ShapeDtypeStruct((NP, 2 * NH), _f32)],
)

_tc_mid = pl.pallas_call(
    _tc_mid_body,
    grid=(NP // ROWB,),
    in_specs=[_part_spec(H), _part_spec(NH), _full_spec(NH, H),
              _full_spec(H, H), _full_spec(H, 2 * NH)],
    out_specs=[_rows_spec(H), _rows_spec(H), _rows_spec(2 * NH)],
    out_shape=[jax.ShapeDtypeStruct((NP, H), _f32),
               jax.ShapeDtypeStruct((NP, H), _f32),
               jax.ShapeDtypeStruct((NP, 2 * NH), _f32)],
)

_tc_fin = pl.pallas_call(
    _tc_fin_body,
    grid=(NP // ROWB,),
    in_specs=[_part_spec(H), _part_spec(NH), _full_spec(NH, H),
              _rows_spec(H)],
    out_specs=_rows_spec(H),
    out_shape=jax.ShapeDtypeStruct((NP, H), _f32),
)


def _build_attn_mat(a_l):
    A = jnp.zeros((H, 2 * NH), _f32)
    for k in range(NH):
        A = A.at[DH * k:DH * (k + 1), 2 * k].set(a_l[k, 0, :DH])
        A = A.at[DH * k:DH * (k + 1), 2 * k + 1].set(a_l[k, 0, DH:])
    return A


def _build_sel():
    S = jnp.zeros((NH, H), _f32)
    for k in range(NH):
        S = S.at[k, DH * k:DH * (k + 1)].set(1.0)
    return S


def _gat_layer_sc(hh, s12, src, dst, vals):
    w, den = _sc_weights(s12.reshape(NP * 8), src, dst, vals)
    (num,) = _sc_aggregate(hh, src, dst, w)
    return num, den.reshape(NC, NP, NH)


def kernel(x, edge_index, edge_vals, W_enc, b_enc, W, a):
    src = edge_index[0]
    dst = edge_index[1]
    x2 = jnp.pad(x[0], ((0, NP - N), (0, 0)))
    b2 = b_enc.reshape(1, H)
    Wc0 = jnp.transpose(W[0], (1, 0, 2)).reshape(H, H)
    Wc1 = jnp.transpose(W[1], (1, 0, 2)).reshape(H, H)
    A0 = _build_attn_mat(a[0])
    A1 = _build_attn_mat(a[1])
    Sel = _build_sel()

    hh0, s12_0 = _tc_enc(x2, W_enc, b2, Wc0, A0)
    num0, den0 = _gat_layer_sc(hh0, s12_0, src, dst, edge_vals)
    h1, hh1, s12_1 = _tc_mid(num0, den0, Sel, Wc1, A1)
    num1, den1 = _gat_layer_sc(hh1, s12_1, src, dst, edge_vals)
    out = _tc_fin(num1, den1, Sel, h1)
    return out[:N].reshape(N, 1, H)


# speedup vs baseline: 22.7681x; 1.0173x over previous
import functools

import jax
import jax.numpy as jnp
from jax import lax
from jax.experimental import pallas as pl
from jax.experimental.pallas import tpu as pltpu
from jax.experimental.pallas import tpu_sc as plsc

N = 10000
NP = 10240
E = 320000
DF = 128
H = 128
NH = 4
DH = H // NH

NC = 2
NS = 16
EPT = E // (NC * NS)
CHA = 2000
NCHA = EPT // CHA
CHB = 80
NCHB = EPT // CHB
RPT = NP // NS

_f32 = jnp.float32
_i32 = jnp.int32

_SC_PARAMS = pltpu.CompilerParams(needs_layout_passes=False)
_SC_MESH = plsc.VectorSubcoreMesh(core_axis_name="c", subcore_axis_name="s")


def _wts_body(s12_hbm, src_hbm, dst_hbm, vals_hbm,
              w_out, den_out,
              s12_v, srcall, dstall, valsall, wbuf, ibuf, den_sh):
    c = lax.axis_index("c")
    s = lax.axis_index("s")

    ebase = (c * NS + s) * EPT
    pltpu.sync_copy(s12_hbm, s12_v)
    pltpu.sync_copy(src_hbm.at[pl.ds(ebase, EPT)], srcall)
    pltpu.sync_copy(dst_hbm.at[pl.ds(ebase, EPT)], dstall)
    pltpu.sync_copy(vals_hbm.at[pl.ds(ebase, EPT)], valsall)

    zlen = NP * NH // NS

    def _z(i, _):
        wbuf[pl.ds(i * 16, 16)] = jnp.zeros((16,), _f32)
        return 0
    lax.fori_loop(0, zlen // 16, _z, 0)
    pltpu.sync_copy(wbuf.at[pl.ds(0, zlen)], den_sh.at[pl.ds(s * zlen, zlen)])
    plsc.subcore_barrier()

    lane = lax.iota(_i32, 16)

    def _chunk(t, _):
        cb = t * CHA

        def _grp(g2, _):
            for u in range(5):
                g = g2 * 5 + u
                off = cb + g * 16
                sv = srcall[pl.ds(off, 16)]
                dv = dstall[pl.ds(off, 16)]
                vv = valsall[pl.ds(off, 16)]
                eid = lane + g * 16
                for k in range(NH):
                    s1 = plsc.load_gather(s12_v, [sv * 8 + (2 * k)])
                    s2 = plsc.load_gather(s12_v, [dv * 8 + (2 * k + 1)])
                    sc = vv * (s1 + s2)
                    sc = jnp.where(sc > 0, sc, 0.2 * sc)
                    w = jnp.exp(sc)
                    pos = eid * NH + k
                    plsc.store_scatter(wbuf, [pos], w)
                    plsc.store_scatter(ibuf, [pos], sv * NH + k)
            return 0
        lax.fori_loop(0, CHA // 80, _grp, 0)

        pltpu.sync_copy(wbuf, w_out.at[pl.ds((ebase + cb) * NH, CHA * NH)])
        pltpu.sync_copy(wbuf, den_sh.at[ibuf], add=True)
        return 0

    lax.fori_loop(0, NCHA, _chunk, 0)
    plsc.subcore_barrier()
    zoff = s * zlen
    pltpu.sync_copy(den_sh.at[pl.ds(zoff, zlen)],
                    den_out.at[c, pl.ds(zoff, zlen)])


_sc_weights = functools.partial(
    pl.kernel,
    out_type=[
        jax.ShapeDtypeStruct((E * NH,), _f32),
        jax.ShapeDtypeStruct((NC, NP * NH), _f32),
    ],
    mesh=_SC_MESH,
    compiler_params=_SC_PARAMS,
    scratch_types=[
        pltpu.VMEM((NP * 8,), _f32),
        pltpu.VMEM((EPT,), _i32),
        pltpu.VMEM((EPT,), _i32),
        pltpu.VMEM((EPT,), _f32),
        pltpu.VMEM((CHA * NH,), _f32),
        pltpu.VMEM((CHA * NH,), _i32),
        pltpu.VMEM_SHARED((NP * NH,), _f32),
    ],
)(_wts_body)


def _agg_body(hh_hbm, src_hbm, dst_hbm, w_hbm,
              num_out,
              dstall, srcb, wv, rows3, num_sh,
              sem_g, sem_s, sem_r, sem_w):
    c = lax.axis_index("c")
    s = lax.axis_index("s")
    wid = c * NS + s
    ebase = wid * EPT
    row0 = s * RPT

    pltpu.sync_copy(dst_hbm.at[pl.ds(ebase, EPT)], dstall)

    def _z(i, _):
        for j in range(H // 16):
            rows3[0, i, pl.ds(j * 16, 16)] = jnp.zeros((16,), _f32)
        return 0
    lax.fori_loop(0, CHB, _z, 0)
    for j in range(RPT // CHB):
        pltpu.sync_copy(rows3.at[0], num_sh.at[pl.ds(row0 + j * CHB, CHB)])

    def _issue_loads(t, j):
        pltpu.async_copy(src_hbm.at[pl.ds(ebase + t * CHB, CHB)],
                         srcb.at[j], sem_r.at[j])
        pltpu.async_copy(
            w_hbm.at[pl.ds((ebase + t * CHB) * NH, CHB * NH)],
            wv.at[pl.ds(j * CHB * NH, CHB * NH)], sem_w.at[j])

    def _wait_loads(t, j):
        pltpu.make_async_copy(src_hbm.at[pl.ds(ebase + t * CHB, CHB)],
                              srcb.at[j], sem_r.at[j]).wait()
        pltpu.make_async_copy(
            w_hbm.at[pl.ds((ebase + t * CHB) * NH, CHB * NH)],
            wv.at[pl.ds(j * CHB * NH, CHB * NH)], sem_w.at[j]).wait()

    def _issue_gather(t, j):
        pltpu.async_copy(hh_hbm.at[dstall.at[pl.ds(t * CHB, CHB)]],
                         rows3.at[j], sem_g.at[j])

    def _wait_gather(t, j):
        pltpu.make_async_copy(hh_hbm.at[dstall.at[pl.ds(t * CHB, CHB)]],
                              rows3.at[j], sem_g.at[j]).wait()

    def _wait_scatter(j):
        pltpu.make_async_copy(rows3.at[j], num_sh.at[srcb.at[j]],
                              sem_s.at[j]).wait()

    def _scale(j):
        wbase = j * CHB * NH

        def _e(e4, _):
            for i in range(4):
                e = e4 * 4 + i
                for k in range(NH):
                    wk = plsc.load_gather(
                        wv, [jnp.full((16,), wbase + e * NH + k, _i32)])
                    for h2 in range(DH // 16):
                        off = k * DH + h2 * 16
                        rows3[j, e, pl.ds(off, 16)] = (
                            rows3[j, e, pl.ds(off, 16)] * wk)
            return 0
        lax.fori_loop(0, CHB // 4, _e, 0)

    def _body(t, j, first=False, last=False):
        j2 = (j + 2) % 3
        _wait_loads(t, j)
        _wait_gather(t, j)
        _scale(j)
        if last:
            pltpu.sync_copy(rows3.at[j], num_sh.at[srcb.at[j]], add=True)
        else:
            pltpu.async_copy(rows3.at[j], num_sh.at[srcb.at[j]],
                             sem_s.at[j], add=True)
        if not first:
            _wait_scatter(j2)
        if not last:
            _issue_loads(t + 2, j2)
            _issue_gather(t + 2, j2)

    _issue_loads(0, 0)
    _issue_loads(1, 1)
    plsc.subcore_barrier()
    _issue_gather(0, 0)
    _issue_gather(1, 1)
    _body(0, 0, first=True)

    def _chunk3(b, _):
        t = 3 * b
        _body(t + 1, 1)
        _body(t + 2, 2)
        _body(t + 3, 0)
        return 0
    lax.fori_loop(0, (NCHB - 5) // 3, _chunk3, 0)

    _body(NCHB - 4, 1)
    _body(NCHB - 3, 2)
    t = NCHB - 2
    _wait_loads(t, 0)
    _wait_gather(t, 0)
    _scale(0)
    pltpu.sync_copy(rows3.at[0], num_sh.at[srcb.at[0]], add=True)
    _wait_scatter(2)
    t = NCHB - 1
    _wait_loads(t, 1)
    _wait_gather(t, 1)
    _scale(1)
    pltpu.sync_copy(rows3.at[1], num_sh.at[srcb.at[1]], add=True)

    plsc.subcore_barrier()
    pltpu.sync_copy(num_sh.at[pl.ds(row0, RPT)],
                    num_out.at[c, pl.ds(row0, RPT)])


_sc_aggregate = functools.partial(
    pl.kernel,
    out_type=[jax.ShapeDtypeStruct((NC, NP, H), _f32)],
    mesh=_SC_MESH,
    compiler_params=_SC_PARAMS,
    scratch_types=[
        pltpu.VMEM((EPT,), _i32),
        pltpu.VMEM((3, CHB), _i32),
        pltpu.VMEM((3 * CHB * NH,), _f32),
        pltpu.VMEM((3, CHB, H), _f32),
        pltpu.VMEM_SHARED((NP, H), _f32),
        pltpu.SemaphoreType.DMA((3,)),
        pltpu.SemaphoreType.DMA((3,)),
        pltpu.SemaphoreType.DMA((3,)),
        pltpu.SemaphoreType.DMA((3,)),
    ],
)(_agg_body)


ROWB = 1280


def _tc_enc_body(x_ref, we_ref, be_ref, wc_ref, a0_ref, hh_ref, s_ref):
    h = jnp.dot(x_ref[...], we_ref[...], preferred_element_type=_f32) + be_ref[...]
    hh = jnp.dot(h, wc_ref[...], preferred_element_type=_f32)
    hh_ref[...] = hh
    s_ref[...] = jnp.dot(hh, a0_ref[...], preferred_element_type=_f32)


def _tc_mid_body(num_ref, den_ref, sel_ref, wc_ref, a1_ref,
                 h1_ref, hh_ref, s_ref):
    nsum = num_ref[0] + num_ref[1]
    dsum = den_ref[0] + den_ref[1]
    dexp = jnp.dot(dsum, sel_ref[...], preferred_element_type=_f32)
    o = nsum / dexp
    h1 = jnp.where(o > 0, o, jnp.exp(o) - 1.0)
    h1_ref[...] = h1
    hh = jnp.dot(h1, wc_ref[...], preferred_element_type=_f32)
    hh_ref[...] = hh
    s_ref[...] = jnp.dot(hh, a1_ref[...], preferred_element_type=_f32)


def _tc_fin_body(num_ref, den_ref, sel_ref, h1_ref, out_ref):
    nsum = num_ref[0] + num_ref[1]
    dsum = den_ref[0] + den_ref[1]
    dexp = jnp.dot(dsum, sel_ref[...], preferred_element_type=_f32)
    out_ref[...] = nsum / dexp + h1_ref[...]


def _rows_spec(width):
    return pl.BlockSpec((ROWB, width), lambda i: (i, 0))


def _full_spec(r, cols):
    return pl.BlockSpec((r, cols), lambda i: (0, 0))


def _part_spec(width):
    return pl.BlockSpec((NC, ROWB, width), lambda i: (0, i, 0))


_tc_enc = pl.pallas_call(
    _tc_enc_body,
    grid=(NP // ROWB,),
    in_specs=[_rows_spec(DF), _full_spec(DF, H), _full_spec(1, H),
              _full_spec(H, H), _full_spec(H, 2 * NH)],
    out_specs=[_rows_spec(H), _rows_spec(2 * NH)],
    out_shape=[jax.ShapeDtypeStruct((NP, H), _f32),
               jax.ShapeDtypeStruct((NP, 2 * NH), _f32)],
)

_tc_mid = pl.pallas_call(
    _tc_mid_body,
    grid=(NP // ROWB,),
    in_specs=[_part_spec(H), _part_spec(NH), _full_spec(NH, H),
              _full_spec(H, H), _full_spec(H, 2 * NH)],
    out_specs=[_rows_spec(H), _rows_spec(H), _rows_spec(2 * NH)],
    out_shape=[jax.ShapeDtypeStruct((NP, H), _f32),
               jax.ShapeDtypeStruct((NP, H), _f32),
               jax.ShapeDtypeStruct((NP, 2 * NH), _f32)],
)

_tc_fin = pl.pallas_call(
    _tc_fin_body,
    grid=(NP // ROWB,),
    in_specs=[_part_spec(H), _part_spec(NH), _full_spec(NH, H),
              _rows_spec(H)],
    out_specs=_rows_spec(H),
    out_shape=jax.ShapeDtypeStruct((NP, H), _f32),
)


def _build_attn_mat(a_l):
    A = jnp.zeros((H, 2 * NH), _f32)
    for k in range(NH):
        A = A.at[DH * k:DH * (k + 1), 2 * k].set(a_l[k, 0, :DH])
        A = A.at[DH * k:DH * (k + 1), 2 * k + 1].set(a_l[k, 0, DH:])
    return A


def _build_sel():
    S = jnp.zeros((NH, H), _f32)
    for k in range(NH):
        S = S.at[k, DH * k:DH * (k + 1)].set(1.0)
    return S


def _gat_layer_sc(hh, s12, src, dst, vals):
    w, den = _sc_weights(s12.reshape(NP * 8), src, dst, vals)
    (num,) = _sc_aggregate(hh, src, dst, w)
    return num, den.reshape(NC, NP, NH)


def kernel(x, edge_index, edge_vals, W_enc, b_enc, W, a):
    src = edge_index[0]
    dst = edge_index[1]
    x2 = jnp.pad(x[0], ((0, NP - N), (0, 0)))
    b2 = b_enc.reshape(1, H)
    Wc0 = jnp.transpose(W[0], (1, 0, 2)).reshape(H, H)
    Wc1 = jnp.transpose(W[1], (1, 0, 2)).reshape(H, H)
    A0 = _build_attn_mat(a[0])
    A1 = _build_attn_mat(a[1])
    Sel = _build_sel()

    hh0, s12_0 = _tc_enc(x2, W_enc, b2, Wc0, A0)
    num0, den0 = _gat_layer_sc(hh0, s12_0, src, dst, edge_vals)
    h1, hh1, s12_1 = _tc_mid(num0, den0, Sel, Wc1, A1)
    num1, den1 = _gat_layer_sc(hh1, s12_1, src, dst, edge_vals)
    out = _tc_fin(num1, den1, Sel, h1)
    return out[:N].reshape(N, 1, H)
